# Initial kernel scaffold; baseline (speedup 1.0000x reference)
import functools

import jax
import jax.numpy as jnp
from jax import lax
from jax.experimental import pallas as pl
from jax.experimental.pallas import tpu as pltpu
from jax.experimental.pallas import tpu_sc as plsc

D = 32


def _build(N, V):
    info = plsc.get_sparse_core_info()
    NC, NS = info.num_cores, info.num_subcores
    NW = NC * NS
    assert N % NW == 0
    b_per_w = N // NW
    CHUNK = 1024
    assert b_per_w % CHUNK == 0
    n_chunks = b_per_w // CHUNK

    mesh = plsc.VectorSubcoreMesh(core_axis_name="c", subcore_axis_name="s")

    @functools.partial(
        pl.kernel,
        mesh=mesh,
        out_type=jax.ShapeDtypeStruct((N, D), jnp.float32),
        scratch_types=[
            pltpu.VMEM((b_per_w,), jnp.int32),
            pltpu.VMEM((CHUNK, D), jnp.float32),
            pltpu.SemaphoreType.DMA,
        ],
    )
    def k(idx_hbm, table_hbm, out_hbm, idx_v, rows_v, gsem):
        wid = lax.axis_index("s") * NC + lax.axis_index("c")
        base = wid * b_per_w
        pltpu.sync_copy(idx_hbm.at[pl.ds(base, b_per_w)], idx_v)

        def body(i, carry):
            off = pl.multiple_of(i * CHUNK, CHUNK)
            pltpu.async_copy(
                table_hbm.at[idx_v.at[pl.ds(off, CHUNK)]], rows_v, gsem
            ).wait()
            pltpu.sync_copy(rows_v, out_hbm.at[pl.ds(base + off, CHUNK)])
            return carry

        lax.fori_loop(0, n_chunks, body, 0)

    return k


def kernel(token_ids, emb):
    B, H = token_ids.shape
    N = B * H
    idx = token_ids.reshape(N).astype(jnp.int32)
    k = _build(N, emb.shape[0])
    out = k(idx, emb)
    return out.reshape(B, H, D)


# speedup vs baseline: 1.1032x; 1.1032x over previous
import functools

import jax
import jax.numpy as jnp
from jax import lax
from jax.experimental import pallas as pl
from jax.experimental.pallas import tpu as pltpu
from jax.experimental.pallas import tpu_sc as plsc

D = 32


def _build(N, V):
    info = plsc.get_sparse_core_info()
    NC, NS = info.num_cores, info.num_subcores
    NW = NC * NS
    assert N % NW == 0
    b_per_w = N // NW
    CHUNK = 1024
    assert b_per_w % CHUNK == 0
    n_chunks = b_per_w // CHUNK

    mesh = plsc.VectorSubcoreMesh(core_axis_name="c", subcore_axis_name="s")

    @functools.partial(
        pl.kernel,
        mesh=mesh,
        out_type=jax.ShapeDtypeStruct((N, D), jnp.float32),
        compiler_params=pltpu.CompilerParams(use_tc_tiling_on_sc=False),
        scratch_types=[
            pltpu.VMEM((b_per_w,), jnp.int32),
            pltpu.VMEM((CHUNK, D), jnp.float32),
            pltpu.SemaphoreType.DMA,
        ],
    )
    def k(idx_hbm, table_hbm, out_hbm, idx_v, rows_v, gsem):
        wid = lax.axis_index("s") * NC + lax.axis_index("c")
        base = wid * b_per_w
        pltpu.sync_copy(idx_hbm.at[pl.ds(base, b_per_w)], idx_v)

        def body(i, carry):
            off = pl.multiple_of(i * CHUNK, CHUNK)
            pltpu.async_copy(
                table_hbm.at[idx_v.at[pl.ds(off, CHUNK)]], rows_v, gsem
            ).wait()
            pltpu.sync_copy(rows_v, out_hbm.at[pl.ds(base + off, CHUNK)])
            return carry

        lax.fori_loop(0, n_chunks, body, 0)

    return k


def kernel(token_ids, emb):
    B, H = token_ids.shape
    N = B * H
    idx = token_ids.reshape(N).astype(jnp.int32)
    k = _build(N, emb.shape[0])
    out = k(idx, emb)
    return out.reshape(B, H, D)


# speedup vs baseline: 1.9304x; 1.7498x over previous
import functools

import jax
import jax.numpy as jnp
from jax import lax
from jax.experimental import pallas as pl
from jax.experimental.pallas import tpu as pltpu
from jax.experimental.pallas import tpu_sc as plsc

D = 32


def _build(N):
    info = plsc.get_sparse_core_info()
    NC, NS = info.num_cores, info.num_subcores
    NW = NC * NS
    assert N % NW == 0
    b_per_w = N // NW
    CHUNK = 1280
    NBUF = 2
    assert b_per_w % (CHUNK * NBUF) == 0
    n_steps = b_per_w // (CHUNK * NBUF)

    mesh = plsc.VectorSubcoreMesh(core_axis_name="c", subcore_axis_name="s")

    @functools.partial(
        pl.kernel,
        mesh=mesh,
        out_type=jax.ShapeDtypeStruct((N, D), jnp.float32),
        compiler_params=pltpu.CompilerParams(use_tc_tiling_on_sc=False),
        scratch_types=[
            pltpu.VMEM((b_per_w,), jnp.int32),
            pltpu.VMEM((NBUF, CHUNK, D), jnp.float32),
            pltpu.SemaphoreType.DMA,
            pltpu.SemaphoreType.DMA,
            pltpu.SemaphoreType.DMA,
            pltpu.SemaphoreType.DMA,
        ],
    )
    def k(idx_hbm, table_hbm, out_hbm, idx_v, rows_v, g0, g1, o0, o1):
        gsem = (g0, g1)
        osem = (o0, o1)
        wid = lax.axis_index("s") * NC + lax.axis_index("c")
        base = wid * b_per_w
        pltpu.sync_copy(idx_hbm.at[pl.ds(base, b_per_w)], idx_v)

        def gather_copy(i, b):
            off = pl.multiple_of(i * CHUNK, CHUNK)
            return pltpu.make_async_copy(
                table_hbm.at[idx_v.at[pl.ds(off, CHUNK)]], rows_v.at[b], gsem[b]
            )

        def store_copy(i, b):
            off = pl.multiple_of(i * CHUNK, CHUNK)
            return pltpu.make_async_copy(
                rows_v.at[b], out_hbm.at[pl.ds(base + off, CHUNK)], osem[b]
            )

        for b in range(NBUF):
            gather_copy(b, b).start()

        def body(s, carry):
            for b in range(NBUF):
                i = s * NBUF + b
                gather_copy(i, b).wait()
                store_copy(i, b).start()
            for b in range(NBUF):
                i = s * NBUF + b
                store_copy(i, b).wait()
                gather_copy(i + NBUF, b).start()
            return carry

        lax.fori_loop(0, n_steps - 1, body, 0)

        for b in range(NBUF):
            i = (n_steps - 1) * NBUF + b
            gather_copy(i, b).wait()
            store_copy(i, b).start()
        for b in range(NBUF):
            i = (n_steps - 1) * NBUF + b
            store_copy(i, b).wait()

    return k


def kernel(token_ids, emb):
    B, H = token_ids.shape
    N = B * H
    idx = token_ids.T.reshape(N).astype(jnp.int32)
    k = _build(N)
    out = k(idx, emb)
    return out.reshape(H, B, D).transpose(1, 0, 2)
